# baseline (device time: 359173 ns/iter reference)
import functools

import jax
import jax.numpy as jnp
from jax import lax
from jax.experimental import pallas as pl
from jax.experimental.pallas import tpu as pltpu

N_DEV = 16
NSUB = 2


def kernel(x, w_mat, scale_x, scale_w):
    m_global, k_per = x.shape
    _, n = w_mat.shape
    m_per = m_global // N_DEV
    half = n // 2
    sub = half // NSUB

    x_bf = x.astype(jnp.bfloat16)
    w_bf = w_mat.astype(jnp.bfloat16)

    def body(x_ref, w_ref, sx_ref, sw_ref, out_ref,
             send_l, recv_l, send_r, recv_r,
             ssem_l, rsem_l, ssem_r, rsem_r, credit_l, credit_r):
        my = lax.axis_index("i")
        left = lax.rem(my + N_DEV - 1, N_DEV)
        right = lax.rem(my + 1, N_DEV)

        def make_rdma(send_buf, recv_buf, ssem, rsem, slot, s, nbr):
            k = slot * NSUB + s
            return pltpu.make_async_remote_copy(
                src_ref=send_buf.at[k],
                dst_ref=recv_buf.at[k],
                send_sem=ssem.at[k],
                recv_sem=rsem.at[k],
                device_id=(nbr,),
                device_id_type=pl.DeviceIdType.MESH,
            )

        part_l = jnp.dot(
            x_ref[pl.ds(lax.rem(my + 1, N_DEV) * m_per, m_per), :],
            w_ref[:, :half], preferred_element_type=jnp.float32)
        part_r = jnp.dot(
            x_ref[pl.ds(lax.rem(my + N_DEV - 1, N_DEV) * m_per, m_per), :],
            w_ref[:, half:], preferred_element_type=jnp.float32)
        for s in range(NSUB):
            send_l[s, :, :] = part_l[:, s * sub:(s + 1) * sub].astype(
                jnp.bfloat16)
            send_r[s, :, :] = part_r[:, s * sub:(s + 1) * sub].astype(
                jnp.bfloat16)

        barrier_sem = pltpu.get_barrier_semaphore()
        for nbr in (left, right):
            pl.semaphore_signal(barrier_sem, inc=1, device_id=(nbr,),
                                device_id_type=pl.DeviceIdType.MESH)
        pl.semaphore_wait(barrier_sem, 2)

        rdmas_l = [[None] * NSUB for _ in range(N_DEV)]
        rdmas_r = [[None] * NSUB for _ in range(N_DEV)]
        for s in range(NSUB):
            for ring_rdmas, send_buf, recv_buf, ssem, rsem, nbr in (
                    (rdmas_l, send_l, recv_l, ssem_l, rsem_l, left),
                    (rdmas_r, send_r, recv_r, ssem_r, rsem_r, right)):
                rdma = make_rdma(send_buf, recv_buf, ssem, rsem, 0, s, nbr)
                rdma.start()
                ring_rdmas[0][s] = rdma

        alpha = sx_ref[0] * sw_ref[0]

        for h in range(1, N_DEV):
            slot = h % 2
            cl = lax.rem(my + h + 1, N_DEV)
            cr = lax.rem(my + 2 * N_DEV - h - 1, N_DEV)
            part_l = jnp.dot(x_ref[pl.ds(cl * m_per, m_per), :],
                             w_ref[:, :half],
                             preferred_element_type=jnp.float32)
            part_r = jnp.dot(x_ref[pl.ds(cr * m_per, m_per), :],
                             w_ref[:, half:],
                             preferred_element_type=jnp.float32)

            if 2 <= h < N_DEV - 1:
                for s in range(NSUB):
                    rdmas_l[h - 2][s].wait_send()
                    rdmas_r[h - 2][s].wait_send()
                pl.semaphore_wait(credit_l, 1)
                pl.semaphore_wait(credit_r, 1)

            for s in range(NSUB):
                for (ring_rdmas, recv_buf, send_buf, ssem, rsem, part, nbr,
                     out_cols) in (
                        (rdmas_l, recv_l, send_l, ssem_l, rsem_l, part_l,
                         left, pl.ds(s * sub, sub)),
                        (rdmas_r, recv_r, send_r, ssem_r, rsem_r, part_r,
                         right, pl.ds(half + s * sub, sub))):
                    ring_rdmas[h - 1][s].wait_recv()
                    rec = recv_buf[((h - 1) % 2) * NSUB + s, :, :]
                    total = (part[:, s * sub:(s + 1) * sub]
                             + rec.astype(jnp.float32))
                    if h < N_DEV - 1:
                        send_buf[slot * NSUB + s, :, :] = (
                            total.astype(jnp.bfloat16))
                        rdma = make_rdma(send_buf, recv_buf, ssem, rsem,
                                         slot, s, nbr)
                        rdma.start()
                        ring_rdmas[h][s] = rdma
                    else:
                        out_ref[:, out_cols] = jnp.maximum(total * alpha, 0.0)

            if 1 <= h <= N_DEV - 3:
                pl.semaphore_signal(credit_l, inc=1, device_id=(right,),
                                    device_id_type=pl.DeviceIdType.MESH)
                pl.semaphore_signal(credit_r, inc=1, device_id=(left,),
                                    device_id_type=pl.DeviceIdType.MESH)

        for h in (N_DEV - 3, N_DEV - 2):
            for s in range(NSUB):
                rdmas_l[h][s].wait_send()
                rdmas_r[h][s].wait_send()

        @functools.partial(pl.run_scoped, sem2=pltpu.SemaphoreType.REGULAR)
        def _(sem2):
            for nbr in (left, right):
                pl.semaphore_signal(sem2, inc=1, device_id=(nbr,),
                                    device_id_type=pl.DeviceIdType.MESH)
            pl.semaphore_wait(sem2, 2)

    return pl.pallas_call(
        body,
        out_shape=jax.ShapeDtypeStruct((m_per, n), jnp.float32),
        in_specs=[
            pl.BlockSpec(memory_space=pltpu.VMEM),
            pl.BlockSpec(memory_space=pltpu.VMEM),
            pl.BlockSpec(memory_space=pltpu.SMEM),
            pl.BlockSpec(memory_space=pltpu.SMEM),
        ],
        out_specs=pl.BlockSpec(memory_space=pltpu.VMEM),
        scratch_shapes=[
            pltpu.VMEM((2 * NSUB, m_per, sub), jnp.bfloat16),
            pltpu.VMEM((2 * NSUB, m_per, sub), jnp.bfloat16),
            pltpu.VMEM((2 * NSUB, m_per, sub), jnp.bfloat16),
            pltpu.VMEM((2 * NSUB, m_per, sub), jnp.bfloat16),
            pltpu.SemaphoreType.DMA((2 * NSUB,)),
            pltpu.SemaphoreType.DMA((2 * NSUB,)),
            pltpu.SemaphoreType.DMA((2 * NSUB,)),
            pltpu.SemaphoreType.DMA((2 * NSUB,)),
            pltpu.SemaphoreType.REGULAR,
            pltpu.SemaphoreType.REGULAR,
        ],
        compiler_params=pltpu.CompilerParams(collective_id=0),
    )(x_bf, w_bf, scale_x, scale_w)


# device time: 359004 ns/iter; 1.0005x vs baseline; 1.0005x over previous
import functools

import jax
import jax.numpy as jnp
from jax import lax
from jax.experimental import pallas as pl
from jax.experimental.pallas import tpu as pltpu

N_DEV = 16
NSUB = 2


def kernel(x, w_mat, scale_x, scale_w):
    m_global, k_per = x.shape
    _, n = w_mat.shape
    m_per = m_global // N_DEV
    half = n // 2
    sub = half // NSUB

    def body(x_in_ref, w_in_ref, sx_ref, sw_ref, out_ref,
             x_ref, w_ref,
             send_l, recv_l, send_r, recv_r,
             ssem_l, rsem_l, ssem_r, rsem_r, credit_l, credit_r):
        my = lax.axis_index("i")
        left = lax.rem(my + N_DEV - 1, N_DEV)
        right = lax.rem(my + 1, N_DEV)

        x_ref[...] = x_in_ref[...].astype(jnp.bfloat16)
        w_ref[...] = w_in_ref[...].astype(jnp.bfloat16)

        def make_rdma(send_buf, recv_buf, ssem, rsem, slot, s, nbr):
            k = slot * NSUB + s
            return pltpu.make_async_remote_copy(
                src_ref=send_buf.at[k],
                dst_ref=recv_buf.at[k],
                send_sem=ssem.at[k],
                recv_sem=rsem.at[k],
                device_id=(nbr,),
                device_id_type=pl.DeviceIdType.MESH,
            )

        part_l = jnp.dot(
            x_ref[pl.ds(lax.rem(my + 1, N_DEV) * m_per, m_per), :],
            w_ref[:, :half], preferred_element_type=jnp.float32)
        part_r = jnp.dot(
            x_ref[pl.ds(lax.rem(my + N_DEV - 1, N_DEV) * m_per, m_per), :],
            w_ref[:, half:], preferred_element_type=jnp.float32)
        for s in range(NSUB):
            send_l[s, :, :] = part_l[:, s * sub:(s + 1) * sub].astype(
                jnp.bfloat16)
            send_r[s, :, :] = part_r[:, s * sub:(s + 1) * sub].astype(
                jnp.bfloat16)

        barrier_sem = pltpu.get_barrier_semaphore()
        for nbr in (left, right):
            pl.semaphore_signal(barrier_sem, inc=1, device_id=(nbr,),
                                device_id_type=pl.DeviceIdType.MESH)
        pl.semaphore_wait(barrier_sem, 2)

        rdmas_l = [[None] * NSUB for _ in range(N_DEV)]
        rdmas_r = [[None] * NSUB for _ in range(N_DEV)]
        for s in range(NSUB):
            for ring_rdmas, send_buf, recv_buf, ssem, rsem, nbr in (
                    (rdmas_l, send_l, recv_l, ssem_l, rsem_l, left),
                    (rdmas_r, send_r, recv_r, ssem_r, rsem_r, right)):
                rdma = make_rdma(send_buf, recv_buf, ssem, rsem, 0, s, nbr)
                rdma.start()
                ring_rdmas[0][s] = rdma

        alpha = sx_ref[0] * sw_ref[0]

        for h in range(1, N_DEV):
            slot = h % 2
            cl = lax.rem(my + h + 1, N_DEV)
            cr = lax.rem(my + 2 * N_DEV - h - 1, N_DEV)
            part_l = jnp.dot(x_ref[pl.ds(cl * m_per, m_per), :],
                             w_ref[:, :half],
                             preferred_element_type=jnp.float32)
            part_r = jnp.dot(x_ref[pl.ds(cr * m_per, m_per), :],
                             w_ref[:, half:],
                             preferred_element_type=jnp.float32)

            if 2 <= h < N_DEV - 1:
                for s in range(NSUB):
                    rdmas_l[h - 2][s].wait_send()
                    rdmas_r[h - 2][s].wait_send()
                pl.semaphore_wait(credit_l, 1)
                pl.semaphore_wait(credit_r, 1)

            for s in range(NSUB):
                for (ring_rdmas, recv_buf, send_buf, ssem, rsem, part, nbr,
                     out_cols) in (
                        (rdmas_l, recv_l, send_l, ssem_l, rsem_l, part_l,
                         left, pl.ds(s * sub, sub)),
                        (rdmas_r, recv_r, send_r, ssem_r, rsem_r, part_r,
                         right, pl.ds(half + s * sub, sub))):
                    ring_rdmas[h - 1][s].wait_recv()
                    rec = recv_buf[((h - 1) % 2) * NSUB + s, :, :]
                    total = (part[:, s * sub:(s + 1) * sub]
                             + rec.astype(jnp.float32))
                    if h < N_DEV - 1:
                        send_buf[slot * NSUB + s, :, :] = (
                            total.astype(jnp.bfloat16))
                        rdma = make_rdma(send_buf, recv_buf, ssem, rsem,
                                         slot, s, nbr)
                        rdma.start()
                        ring_rdmas[h][s] = rdma
                    else:
                        out_ref[:, out_cols] = jnp.maximum(total * alpha, 0.0)

            if 1 <= h <= N_DEV - 3:
                pl.semaphore_signal(credit_l, inc=1, device_id=(right,),
                                    device_id_type=pl.DeviceIdType.MESH)
                pl.semaphore_signal(credit_r, inc=1, device_id=(left,),
                                    device_id_type=pl.DeviceIdType.MESH)

        for h in (N_DEV - 3, N_DEV - 2):
            for s in range(NSUB):
                rdmas_l[h][s].wait_send()
                rdmas_r[h][s].wait_send()

        @functools.partial(pl.run_scoped, sem2=pltpu.SemaphoreType.REGULAR)
        def _(sem2):
            for nbr in (left, right):
                pl.semaphore_signal(sem2, inc=1, device_id=(nbr,),
                                    device_id_type=pl.DeviceIdType.MESH)
            pl.semaphore_wait(sem2, 2)

    return pl.pallas_call(
        body,
        out_shape=jax.ShapeDtypeStruct((m_per, n), jnp.float32),
        in_specs=[
            pl.BlockSpec(memory_space=pltpu.VMEM),
            pl.BlockSpec(memory_space=pltpu.VMEM),
            pl.BlockSpec(memory_space=pltpu.SMEM),
            pl.BlockSpec(memory_space=pltpu.SMEM),
        ],
        out_specs=pl.BlockSpec(memory_space=pltpu.VMEM),
        scratch_shapes=[
            pltpu.VMEM((m_global, k_per), jnp.bfloat16),
            pltpu.VMEM((k_per, n), jnp.bfloat16),
            pltpu.VMEM((2 * NSUB, m_per, sub), jnp.bfloat16),
            pltpu.VMEM((2 * NSUB, m_per, sub), jnp.bfloat16),
            pltpu.VMEM((2 * NSUB, m_per, sub), jnp.bfloat16),
            pltpu.VMEM((2 * NSUB, m_per, sub), jnp.bfloat16),
            pltpu.SemaphoreType.DMA((2 * NSUB,)),
            pltpu.SemaphoreType.DMA((2 * NSUB,)),
            pltpu.SemaphoreType.DMA((2 * NSUB,)),
            pltpu.SemaphoreType.DMA((2 * NSUB,)),
            pltpu.SemaphoreType.REGULAR,
            pltpu.SemaphoreType.REGULAR,
        ],
        compiler_params=pltpu.CompilerParams(collective_id=0),
    )(x, w_mat, scale_x, scale_w)


# device time: 358868 ns/iter; 1.0008x vs baseline; 1.0004x over previous
import functools

import jax
import jax.numpy as jnp
from jax import lax
from jax.experimental import pallas as pl
from jax.experimental.pallas import tpu as pltpu

N_DEV = 16
NSUB = 4


def kernel(x, w_mat, scale_x, scale_w):
    m_global, k_per = x.shape
    _, n = w_mat.shape
    m_per = m_global // N_DEV
    half = n // 2
    sub = half // NSUB

    def body(x_in_ref, w_in_ref, sx_ref, sw_ref, out_ref,
             x_ref, w_ref,
             send_l, recv_l, send_r, recv_r,
             ssem_l, rsem_l, ssem_r, rsem_r, credit_l, credit_r):
        my = lax.axis_index("i")
        left = lax.rem(my + N_DEV - 1, N_DEV)
        right = lax.rem(my + 1, N_DEV)

        x_ref[...] = x_in_ref[...].astype(jnp.bfloat16)
        w_ref[...] = w_in_ref[...].astype(jnp.bfloat16)

        def make_rdma(send_buf, recv_buf, ssem, rsem, slot, s, nbr):
            k = slot * NSUB + s
            return pltpu.make_async_remote_copy(
                src_ref=send_buf.at[k],
                dst_ref=recv_buf.at[k],
                send_sem=ssem.at[k],
                recv_sem=rsem.at[k],
                device_id=(nbr,),
                device_id_type=pl.DeviceIdType.MESH,
            )

        part_l = jnp.dot(
            x_ref[pl.ds(lax.rem(my + 1, N_DEV) * m_per, m_per), :],
            w_ref[:, :half], preferred_element_type=jnp.float32)
        part_r = jnp.dot(
            x_ref[pl.ds(lax.rem(my + N_DEV - 1, N_DEV) * m_per, m_per), :],
            w_ref[:, half:], preferred_element_type=jnp.float32)
        for s in range(NSUB):
            send_l[s, :, :] = part_l[:, s * sub:(s + 1) * sub].astype(
                jnp.bfloat16)
            send_r[s, :, :] = part_r[:, s * sub:(s + 1) * sub].astype(
                jnp.bfloat16)

        barrier_sem = pltpu.get_barrier_semaphore()
        for nbr in (left, right):
            pl.semaphore_signal(barrier_sem, inc=1, device_id=(nbr,),
                                device_id_type=pl.DeviceIdType.MESH)
        pl.semaphore_wait(barrier_sem, 2)

        rdmas_l = [[None] * NSUB for _ in range(N_DEV)]
        rdmas_r = [[None] * NSUB for _ in range(N_DEV)]
        for s in range(NSUB):
            for ring_rdmas, send_buf, recv_buf, ssem, rsem, nbr in (
                    (rdmas_l, send_l, recv_l, ssem_l, rsem_l, left),
                    (rdmas_r, send_r, recv_r, ssem_r, rsem_r, right)):
                rdma = make_rdma(send_buf, recv_buf, ssem, rsem, 0, s, nbr)
                rdma.start()
                ring_rdmas[0][s] = rdma

        alpha = sx_ref[0] * sw_ref[0]

        for h in range(1, N_DEV):
            slot = h % 2
            cl = lax.rem(my + h + 1, N_DEV)
            cr = lax.rem(my + 2 * N_DEV - h - 1, N_DEV)
            part_l = jnp.dot(x_ref[pl.ds(cl * m_per, m_per), :],
                             w_ref[:, :half],
                             preferred_element_type=jnp.float32)
            part_r = jnp.dot(x_ref[pl.ds(cr * m_per, m_per), :],
                             w_ref[:, half:],
                             preferred_element_type=jnp.float32)

            if 2 <= h < N_DEV - 1:
                for s in range(NSUB):
                    rdmas_l[h - 2][s].wait_send()
                    rdmas_r[h - 2][s].wait_send()
                pl.semaphore_wait(credit_l, 1)
                pl.semaphore_wait(credit_r, 1)

            for s in range(NSUB):
                for (ring_rdmas, recv_buf, send_buf, ssem, rsem, part, nbr,
                     out_cols) in (
                        (rdmas_l, recv_l, send_l, ssem_l, rsem_l, part_l,
                         left, pl.ds(s * sub, sub)),
                        (rdmas_r, recv_r, send_r, ssem_r, rsem_r, part_r,
                         right, pl.ds(half + s * sub, sub))):
                    ring_rdmas[h - 1][s].wait_recv()
                    rec = recv_buf[((h - 1) % 2) * NSUB + s, :, :]
                    total = (part[:, s * sub:(s + 1) * sub]
                             + rec.astype(jnp.float32))
                    if h < N_DEV - 1:
                        send_buf[slot * NSUB + s, :, :] = (
                            total.astype(jnp.bfloat16))
                        rdma = make_rdma(send_buf, recv_buf, ssem, rsem,
                                         slot, s, nbr)
                        rdma.start()
                        ring_rdmas[h][s] = rdma
                    else:
                        out_ref[:, out_cols] = jnp.maximum(total * alpha, 0.0)

            if 1 <= h <= N_DEV - 3:
                pl.semaphore_signal(credit_l, inc=1, device_id=(right,),
                                    device_id_type=pl.DeviceIdType.MESH)
                pl.semaphore_signal(credit_r, inc=1, device_id=(left,),
                                    device_id_type=pl.DeviceIdType.MESH)

        for h in (N_DEV - 3, N_DEV - 2):
            for s in range(NSUB):
                rdmas_l[h][s].wait_send()
                rdmas_r[h][s].wait_send()

        @functools.partial(pl.run_scoped, sem2=pltpu.SemaphoreType.REGULAR)
        def _(sem2):
            for nbr in (left, right):
                pl.semaphore_signal(sem2, inc=1, device_id=(nbr,),
                                    device_id_type=pl.DeviceIdType.MESH)
            pl.semaphore_wait(sem2, 2)

    return pl.pallas_call(
        body,
        out_shape=jax.ShapeDtypeStruct((m_per, n), jnp.float32),
        in_specs=[
            pl.BlockSpec(memory_space=pltpu.VMEM),
            pl.BlockSpec(memory_space=pltpu.VMEM),
            pl.BlockSpec(memory_space=pltpu.SMEM),
            pl.BlockSpec(memory_space=pltpu.SMEM),
        ],
        out_specs=pl.BlockSpec(memory_space=pltpu.VMEM),
        scratch_shapes=[
            pltpu.VMEM((m_global, k_per), jnp.bfloat16),
            pltpu.VMEM((k_per, n), jnp.bfloat16),
            pltpu.VMEM((2 * NSUB, m_per, sub), jnp.bfloat16),
            pltpu.VMEM((2 * NSUB, m_per, sub), jnp.bfloat16),
            pltpu.VMEM((2 * NSUB, m_per, sub), jnp.bfloat16),
            pltpu.VMEM((2 * NSUB, m_per, sub), jnp.bfloat16),
            pltpu.SemaphoreType.DMA((2 * NSUB,)),
            pltpu.SemaphoreType.DMA((2 * NSUB,)),
            pltpu.SemaphoreType.DMA((2 * NSUB,)),
            pltpu.SemaphoreType.DMA((2 * NSUB,)),
            pltpu.SemaphoreType.REGULAR,
            pltpu.SemaphoreType.REGULAR,
        ],
        compiler_params=pltpu.CompilerParams(collective_id=0),
    )(x, w_mat, scale_x, scale_w)
